# Initial kernel scaffold; baseline (speedup 1.0000x reference)
import jax
import jax.numpy as jnp
from jax.experimental import pallas as pl


def kernel(src, indices):
    raise NotImplementedError("write your pallas kernel here")


# speedup vs baseline: 8.1959x; 8.1959x over previous
import jax
import jax.numpy as jnp
from jax.experimental import pallas as pl
from jax.experimental.pallas import tpu as pltpu

N = 4096
GC_BLK = 512
M_BLK = 512
N_BLK = 2048


def _build_g_kernel(idx_ref, g_ref):
    idx = idx_ref[...]
    i = jnp.floor(idx)
    f = idx - i
    ii = i.astype(jnp.int32)
    f2 = f * f
    f3 = f2 * f
    w0 = -0.5 * f + f2 - 0.5 * f3
    w1 = 1.0 - 2.5 * f2 + 1.5 * f3
    w2 = 0.5 * f + 2.0 * f2 - 1.5 * f3
    w3 = -0.5 * f2 + 0.5 * f3
    c0 = pl.program_id(0) * GC_BLK
    c = jax.lax.broadcasted_iota(jnp.int32, (GC_BLK, N), 0) + c0
    g = jnp.zeros((GC_BLK, N), jnp.float32)
    for t, w in ((-1, w0), (0, w1), (1, w2), (2, w3)):
        tap = jnp.clip(ii + t, 0, N - 1)
        g = g + jnp.where(c == tap, w, 0.0)
    g_ref[...] = g.astype(jnp.bfloat16)


def _matmul_kernel(a_ref, g_ref, o_ref):
    o_ref[...] = jnp.dot(a_ref[...], g_ref[...],
                         preferred_element_type=jnp.float32)


def kernel(src, indices):
    idx2d = indices.reshape(1, N)
    g = pl.pallas_call(
        _build_g_kernel,
        grid=(N // GC_BLK,),
        in_specs=[pl.BlockSpec((1, N), lambda c: (0, 0))],
        out_specs=pl.BlockSpec((GC_BLK, N), lambda c: (c, 0)),
        out_shape=jax.ShapeDtypeStruct((N, N), jnp.bfloat16),
    )(idx2d)

    src_bf = src.astype(jnp.bfloat16)
    out = pl.pallas_call(
        _matmul_kernel,
        grid=(N // N_BLK, N // M_BLK),
        in_specs=[
            pl.BlockSpec((M_BLK, N), lambda n, m: (m, 0)),
            pl.BlockSpec((N, N_BLK), lambda n, m: (0, n)),
        ],
        out_specs=pl.BlockSpec((M_BLK, N_BLK), lambda n, m: (m, n)),
        out_shape=jax.ShapeDtypeStruct((N, N), jnp.float32),
        compiler_params=pltpu.CompilerParams(
            dimension_semantics=("arbitrary", "arbitrary"),
        ),
    )(src_bf, g)
    return out


# speedup vs baseline: 32.1512x; 3.9228x over previous
import jax
import jax.numpy as jnp
from jax.experimental import pallas as pl
from jax.experimental.pallas import tpu as pltpu

N = 4096
KB = 512
NKB = N // KB
M_BLK = 512


def _build_g_kernel(idx_ref, g_ref, meta_ref, sm_ref):
    k = pl.program_id(0)

    @pl.when(k == 0)
    def _():
        idx = idx_ref[...]
        tap_min = jnp.floor(jnp.min(idx)).astype(jnp.int32) - 1
        tap_min = jnp.maximum(tap_min, 0)
        tap_max = jnp.floor(jnp.max(idx)).astype(jnp.int32) + 2
        tap_max = jnp.minimum(tap_max, N - 1)
        base = tap_min >> 9
        nblk = (tap_max >> 9) - base + 1
        sm_ref[0] = base
        sm_ref[1] = nblk
        lane = jax.lax.broadcasted_iota(jnp.int32, (1, 128), 1)
        meta_ref[...] = jnp.where(lane == 0, base, jnp.where(lane == 1, nblk, 0))

    base = sm_ref[0]
    nblk = sm_ref[1]

    @pl.when(k < nblk)
    def _():
        idx = idx_ref[...]
        i = jnp.floor(idx)
        f = idx - i
        ii = i.astype(jnp.int32)
        f2 = f * f
        f3 = f2 * f
        w0 = -0.5 * f + f2 - 0.5 * f3
        w1 = 1.0 - 2.5 * f2 + 1.5 * f3
        w2 = 0.5 * f + 2.0 * f2 - 1.5 * f3
        w3 = -0.5 * f2 + 0.5 * f3
        c0 = (base + k) * KB
        c = jax.lax.broadcasted_iota(jnp.int32, (KB, N), 0) + c0
        g = jnp.zeros((KB, N), jnp.float32)
        for t, w in ((-1, w0), (0, w1), (1, w2), (2, w3)):
            tap = jnp.clip(ii + t, 0, N - 1)
            g = g + jnp.where(c == tap, w, 0.0)
        g_ref[...] = g.astype(jnp.bfloat16)


def _matmul_kernel(meta_ref, a_ref, g_ref, o_ref, acc_ref):
    k = pl.program_id(1)
    nblk = meta_ref[1]

    @pl.when(k == 0)
    def _():
        acc_ref[...] = jnp.dot(a_ref[...].astype(jnp.bfloat16), g_ref[...],
                               preferred_element_type=jnp.float32)

    @pl.when((k > 0) & (k < nblk))
    def _():
        acc_ref[...] += jnp.dot(a_ref[...].astype(jnp.bfloat16), g_ref[...],
                                preferred_element_type=jnp.float32)

    @pl.when(k == NKB - 1)
    def _():
        o_ref[...] = acc_ref[...]


def kernel(src, indices):
    idx2d = indices.reshape(1, N)

    def gb_out_map(k):
        return (k, 0)

    g, meta = pl.pallas_call(
        _build_g_kernel,
        grid=(NKB,),
        in_specs=[pl.BlockSpec((1, N), lambda k: (0, 0))],
        out_specs=[
            pl.BlockSpec((KB, N), gb_out_map),
            pl.BlockSpec((1, 128), lambda k: (0, 0)),
        ],
        out_shape=[
            jax.ShapeDtypeStruct((N, N), jnp.bfloat16),
            jax.ShapeDtypeStruct((1, 128), jnp.int32),
        ],
        scratch_shapes=[pltpu.SMEM((2,), jnp.int32)],
        compiler_params=pltpu.CompilerParams(
            dimension_semantics=("arbitrary",),
        ),
    )(idx2d)

    meta1d = meta.reshape(128)

    def kk(k, meta_ref):
        return jnp.minimum(k, meta_ref[1] - 1)

    out = pl.pallas_call(
        _matmul_kernel,
        grid_spec=pltpu.PrefetchScalarGridSpec(
            num_scalar_prefetch=1,
            grid=(N // M_BLK, NKB),
            in_specs=[
                pl.BlockSpec((M_BLK, KB),
                             lambda m, k, meta: (m, meta[0] + kk(k, meta))),
                pl.BlockSpec((KB, N),
                             lambda m, k, meta: (kk(k, meta), 0)),
            ],
            out_specs=pl.BlockSpec((M_BLK, N), lambda m, k, meta: (m, 0)),
            scratch_shapes=[pltpu.VMEM((M_BLK, N), jnp.float32)],
        ),
        out_shape=jax.ShapeDtypeStruct((N, N), jnp.float32),
        compiler_params=pltpu.CompilerParams(
            dimension_semantics=("arbitrary", "arbitrary"),
        ),
    )(meta1d, src, g)
    return out
